# Initial kernel scaffold; baseline (speedup 1.0000x reference)
import jax
import jax.numpy as jnp
from jax.experimental import pallas as pl


def kernel(norm_data, Wg, W1, W2):
    raise NotImplementedError("write your pallas kernel here")


# speedup vs baseline: 1.0052x; 1.0052x over previous
import functools

import jax
import jax.numpy as jnp
from jax import lax
from jax.experimental import pallas as pl
from jax.experimental.pallas import tpu as pltpu
from jax.experimental.pallas import tpu_sc as plsc

S = 2048
D = 1024
E = 8
DFF = 2048
CAP = 384
SLOTS = E * CAP
LANEPAD = 128

NC = 2
NS = 16
NW = NC * NS
CHUNK = SLOTS // NW
TOK_W = S // NW
HALF = TOK_W // 2

_f32 = jnp.float32
_i32 = jnp.int32


def _router_body(x_ref, wg_ref, logits_ref, pm_ref, idx_ref, onehot_ref):
    x = x_ref[...]
    wg = wg_ref[...]
    logits = jnp.dot(x, wg, preferred_element_type=_f32)
    logits_ref[...] = logits
    lane = lax.broadcasted_iota(_i32, (S, LANEPAD), 1)
    ml = jnp.where(lane < E, logits, _f32(-1e30))
    mx = jnp.max(ml, axis=1, keepdims=True)
    ex = jnp.exp(ml - mx)
    ssum = jnp.sum(ex, axis=1, keepdims=True)
    probs = ex / ssum
    pm = jnp.max(probs, axis=1, keepdims=True)
    pm_ref[...] = pm
    cand = jnp.where((probs == pm) & (lane < E), lane, _i32(LANEPAD - 1))
    idx = jnp.min(cand, axis=1, keepdims=True)
    idx_ref[...] = idx
    onehot_ref[...] = (lane == idx).astype(jnp.bfloat16)


_router = pl.pallas_call(
    _router_body,
    out_shape=[
        jax.ShapeDtypeStruct((S, LANEPAD), _f32),
        jax.ShapeDtypeStruct((S, 1), _f32),
        jax.ShapeDtypeStruct((S, 1), _i32),
        jax.ShapeDtypeStruct((S, LANEPAD), jnp.bfloat16),
    ],
)


_PRIO_ROWS = 128


def _prio_body(onehot_ref, idx_ref, slotbt_ref, routed_ref, eidx_ref):
    i = pl.program_id(0)
    r = lax.broadcasted_iota(_i32, (_PRIO_ROWS, S), 0) + i * _PRIO_ROWS
    c = lax.broadcasted_iota(_i32, (_PRIO_ROWS, S), 1)
    ltm = (r >= c).astype(jnp.bfloat16)
    prio = jnp.dot(ltm, onehot_ref[...], preferred_element_type=_f32)
    lane = lax.broadcasted_iota(_i32, (_PRIO_ROWS, LANEPAD), 1)
    idx = idx_ref[...]
    p = jnp.sum(jnp.where(lane == idx, prio, 0.0), axis=1,
                keepdims=True).astype(_i32)
    routed = p <= CAP
    slot = idx * CAP + jnp.minimum(p, CAP) - 1
    slotbt_ref[...] = jnp.where(routed, slot, 0)
    routed_ref[...] = routed.astype(_i32)
    eidx_ref[...] = jnp.where(routed, idx, 0)


_priority = pl.pallas_call(
    _prio_body,
    grid=(S // _PRIO_ROWS,),
    in_specs=[
        pl.BlockSpec((S, LANEPAD), lambda i: (0, 0)),
        pl.BlockSpec((_PRIO_ROWS, 1), lambda i: (i, 0)),
    ],
    out_specs=[
        pl.BlockSpec((_PRIO_ROWS, 1), lambda i: (i, 0)),
        pl.BlockSpec((_PRIO_ROWS, 1), lambda i: (i, 0)),
        pl.BlockSpec((_PRIO_ROWS, 1), lambda i: (i, 0)),
    ],
    out_shape=[
        jax.ShapeDtypeStruct((S, 1), _i32),
        jax.ShapeDtypeStruct((S, 1), _i32),
        jax.ShapeDtypeStruct((S, 1), _i32),
    ],
)


_mesh = plsc.VectorSubcoreMesh(core_axis_name="c", subcore_axis_name="s",
                               num_cores=NC, num_subcores=NS)


@functools.partial(
    pl.kernel,
    out_type=jax.ShapeDtypeStruct((SLOTS, D), _f32),
    mesh=_mesh,
    scratch_types=[
        pltpu.VMEM((S,), _i32),
        pltpu.VMEM((S,), _i32),
        pltpu.VMEM((SLOTS,), _i32),
        pltpu.VMEM((CHUNK,), _i32),
        pltpu.VMEM((CHUNK, D), _f32),
        pltpu.SemaphoreType.DMA,
    ],
    compiler_params=pltpu.CompilerParams(needs_layout_passes=False),
)
def _dispatch(slotbt_hbm, routed_hbm, x_hbm, xbuf_hbm,
              sbt_v, rt_v, st_v, idx_v, rows_v, sem):
    pltpu.sync_copy(slotbt_hbm, sbt_v)
    pltpu.sync_copy(routed_hbm, rt_v)

    def init_body(i, _):
        st_v[pl.ds(i * 16, 16)] = jnp.zeros((16,), _i32)
        return 0

    lax.fori_loop(0, SLOTS // 16, init_body, 0)

    def inv_body(i, _):
        s = sbt_v[pl.ds(i * 16, 16)]
        r = rt_v[pl.ds(i * 16, 16)]
        toks = lax.iota(_i32, 16) + i * 16
        plsc.store_scatter(st_v, [s], toks, mask=r > 0)
        return 0

    lax.fori_loop(0, S // 16, inv_body, 0)

    wid = lax.axis_index("s") * NC + lax.axis_index("c")
    base = wid * CHUNK
    for k in range(CHUNK // 16):
        idx_v[pl.ds(k * 16, 16)] = st_v[pl.ds(base + k * 16, 16)]
    pltpu.async_copy(x_hbm.at[idx_v], rows_v, sem).wait()
    pltpu.sync_copy(rows_v, xbuf_hbm.at[pl.ds(base, CHUNK)])


def _ffn_body(xb_ref, w1_ref, w2_ref, o_ref):
    h = jnp.dot(xb_ref[...], w1_ref[0], preferred_element_type=_f32)
    h = jnp.maximum(h, 0.0)
    o_ref[...] = jnp.dot(h, w2_ref[0], preferred_element_type=_f32)


_ffn = pl.pallas_call(
    _ffn_body,
    grid=(E,),
    in_specs=[
        pl.BlockSpec((CAP, D), lambda e: (e, 0)),
        pl.BlockSpec((1, D, DFF), lambda e: (e, 0, 0)),
        pl.BlockSpec((1, DFF, D), lambda e: (e, 0, 0)),
    ],
    out_specs=pl.BlockSpec((CAP, D), lambda e: (e, 0)),
    out_shape=jax.ShapeDtypeStruct((SLOTS, D), _f32),
)


@functools.partial(
    pl.kernel,
    out_type=jax.ShapeDtypeStruct((S, D), _f32),
    mesh=_mesh,
    scratch_types=[
        pltpu.VMEM((TOK_W,), _i32),
        pltpu.VMEM((TOK_W,), _i32),
        pltpu.VMEM((TOK_W,), _f32),
        pltpu.VMEM((TOK_W + 16,), _f32),
        pltpu.VMEM((TOK_W + 16,), _f32),
        pltpu.VMEM((HALF, D), _f32),
        pltpu.VMEM((HALF, D), _f32),
        pltpu.VMEM((HALF, D), _f32),
        pltpu.SemaphoreType.DMA,
    ],
    compiler_params=pltpu.CompilerParams(needs_layout_passes=False),
)
def _combine(slotbt_hbm, routed_hbm, pm_hbm, x_hbm, ffn_hbm, out_hbm,
             sbt_v, rt_v, pm_v, a_v, b_v, f_v, x_v, o_v, sem):
    wid = lax.axis_index("s") * NC + lax.axis_index("c")
    tbase = wid * TOK_W
    pltpu.sync_copy(slotbt_hbm.at[pl.ds(tbase, TOK_W)], sbt_v)
    pltpu.sync_copy(routed_hbm.at[pl.ds(tbase, TOK_W)], rt_v)
    pltpu.sync_copy(pm_hbm.at[pl.ds(tbase, TOK_W)], pm_v)
    for g in range(TOK_W // 16):
        sl = pl.ds(g * 16, 16)
        m = rt_v[sl] > 0
        p = pm_v[sl]
        a_v[sl] = jnp.where(m, p, 0.0)
        b_v[sl] = jnp.where(m, 0.0, p)
    a_v[pl.ds(TOK_W, 16)] = jnp.zeros((16,), _f32)
    b_v[pl.ds(TOK_W, 16)] = jnp.zeros((16,), _f32)
    for h in range(2):
        pltpu.async_copy(ffn_hbm.at[sbt_v.at[pl.ds(h * HALF, HALF)]],
                         f_v, sem).wait()
        pltpu.sync_copy(x_hbm.at[pl.ds(tbase + h * HALF, HALF)], x_v)

        def jbody(j, _):
            aj = a_v[pl.ds(h * HALF + j, 16)][0]
            bj = b_v[pl.ds(h * HALF + j, 16)][0]

            def kbody(k, _):
                ff = f_v[j, pl.ds(k * 16, 16)]
                xx = x_v[j, pl.ds(k * 16, 16)]
                o_v[j, pl.ds(k * 16, 16)] = aj * ff + bj * xx
                return 0

            lax.fori_loop(0, D // 16, kbody, 0)
            return 0

        lax.fori_loop(0, HALF, jbody, 0)
        pltpu.sync_copy(o_v, out_hbm.at[pl.ds(tbase + h * HALF, HALF)])


def kernel(norm_data, Wg, W1, W2):
    x = norm_data.reshape(S, D).astype(_f32)
    wgp = jnp.pad(Wg.astype(_f32), ((0, 0), (0, LANEPAD - E)))
    logits128, pm, idx, onehot = _router(x, wgp)
    slotbt, routed, eidx = _priority(onehot, idx)
    router_logits = logits128[:, :E].reshape(1, S, E)
    slotbt1 = slotbt.reshape(S)
    routed1 = routed.reshape(S)
    pm1 = pm.reshape(S)
    xbuf = _dispatch(slotbt1, routed1, x)
    ffnbuf = _ffn(xbuf, W1, W2)
    out = _combine(slotbt1, routed1, pm1, x, ffnbuf)
    return (out.reshape(1, S, D), router_logits, eidx.reshape(1, S))


# speedup vs baseline: 1.1769x; 1.1708x over previous
import functools

import jax
import jax.numpy as jnp
from jax import lax
from jax.experimental import pallas as pl
from jax.experimental.pallas import tpu as pltpu
from jax.experimental.pallas import tpu_sc as plsc

S = 2048
D = 1024
E = 8
DFF = 2048
CAP = 384
SLOTS = E * CAP
KEEP_PAD = 2304
SRC_ROWS = SLOTS + KEEP_PAD
LANEPAD = 128

NC = 2
NS = 16
NW = NC * NS
CHUNK = SLOTS // NW
TOK_T = S // NS
TOK_W = S // NW

_f32 = jnp.float32
_i32 = jnp.int32


def _router_body(x_ref, wg_ref, logits_ref, pm_ref, idx_ref, onehot_ref):
    x = x_ref[...]
    wg = wg_ref[...]
    logits = jnp.dot(x, wg, preferred_element_type=_f32)
    logits_ref[...] = logits
    lane = lax.broadcasted_iota(_i32, (S, LANEPAD), 1)
    ml = jnp.where(lane < E, logits, _f32(-1e30))
    mx = jnp.max(ml, axis=1, keepdims=True)
    ex = jnp.exp(ml - mx)
    ssum = jnp.sum(ex, axis=1, keepdims=True)
    probs = ex / ssum
    pm = jnp.max(probs, axis=1, keepdims=True)
    pm_ref[...] = pm
    cand = jnp.where((probs == pm) & (lane < E), lane, _i32(LANEPAD - 1))
    idx = jnp.min(cand, axis=1, keepdims=True)
    idx_ref[...] = idx
    onehot_ref[...] = (lane == idx).astype(jnp.bfloat16)


_router = pl.pallas_call(
    _router_body,
    out_shape=[
        jax.ShapeDtypeStruct((S, LANEPAD), _f32),
        jax.ShapeDtypeStruct((S, 1), _f32),
        jax.ShapeDtypeStruct((S, 1), _i32),
        jax.ShapeDtypeStruct((S, LANEPAD), jnp.bfloat16),
    ],
)


_PRIO_ROWS = 128


def _prio_body(onehot_ref, idx_ref, src_ref, eidx_ref):
    i = pl.program_id(0)
    r = lax.broadcasted_iota(_i32, (_PRIO_ROWS, S), 0) + i * _PRIO_ROWS
    c = lax.broadcasted_iota(_i32, (_PRIO_ROWS, S), 1)
    ltm = (r >= c).astype(jnp.bfloat16)
    prio = jnp.dot(ltm, onehot_ref[...], preferred_element_type=_f32)
    lane = lax.broadcasted_iota(_i32, (_PRIO_ROWS, LANEPAD), 1)
    idx = idx_ref[...]
    p = jnp.sum(jnp.where(lane == idx, prio, 0.0), axis=1,
                keepdims=True).astype(_i32)
    routed = p <= CAP
    tok = lax.broadcasted_iota(_i32, (_PRIO_ROWS, 1), 0) + i * _PRIO_ROWS
    slot = idx * CAP + jnp.minimum(p, CAP) - 1
    src_ref[...] = jnp.where(routed, slot, SLOTS + tok)
    eidx_ref[...] = jnp.where(routed, idx, 0)


_priority = pl.pallas_call(
    _prio_body,
    grid=(S // _PRIO_ROWS,),
    in_specs=[
        pl.BlockSpec((S, LANEPAD), lambda i: (0, 0)),
        pl.BlockSpec((_PRIO_ROWS, 1), lambda i: (i, 0)),
    ],
    out_specs=[
        pl.BlockSpec((_PRIO_ROWS, 1), lambda i: (i, 0)),
        pl.BlockSpec((_PRIO_ROWS, 1), lambda i: (i, 0)),
    ],
    out_shape=[
        jax.ShapeDtypeStruct((S, 1), _i32),
        jax.ShapeDtypeStruct((S, 1), _i32),
    ],
)


_mesh = plsc.VectorSubcoreMesh(core_axis_name="c", subcore_axis_name="s",
                               num_cores=NC, num_subcores=NS)


@functools.partial(
    pl.kernel,
    out_type=(
        jax.ShapeDtypeStruct((SLOTS, D), _f32),
        jax.ShapeDtypeStruct((SLOTS,), _f32),
    ),
    mesh=_mesh,
    scratch_types=[
        pltpu.VMEM((TOK_T,), _i32),
        pltpu.VMEM((TOK_T,), _i32),
        pltpu.VMEM((CHUNK,), _i32),
        pltpu.VMEM((CHUNK, D), _f32),
        pltpu.VMEM((CHUNK,), _f32),
        pltpu.VMEM_SHARED((SRC_ROWS,), _i32),
        pltpu.SemaphoreType.DMA,
        pltpu.SemaphoreType.DMA,
    ],
    compiler_params=pltpu.CompilerParams(needs_layout_passes=False),
)
def _dispatch(src_hbm, pm_hbm, x_hbm, xbuf_hbm, pms_hbm,
              sidx_v, tok_v, cidx_v, rows_v, pmc_v, st_sh, sem, sem2):
    sid = lax.axis_index("s")
    tb = sid * TOK_T
    pltpu.sync_copy(src_hbm.at[pl.ds(tb, TOK_T)], sidx_v)
    for g in range(TOK_T // 16):
        tok_v[pl.ds(g * 16, 16)] = lax.iota(_i32, 16) + (tb + g * 16)
    pltpu.sync_copy(tok_v, st_sh.at[sidx_v])
    plsc.subcore_barrier()
    wid = sid * NC + lax.axis_index("c")
    cbase = wid * CHUNK
    pltpu.sync_copy(st_sh.at[pl.ds(cbase, CHUNK)], cidx_v)
    for g in range(CHUNK // 16):
        sl = pl.ds(g * 16, 16)
        v = cidx_v[sl]
        cidx_v[sl] = jnp.minimum(jnp.maximum(v, 0), S - 1)
    d1 = pltpu.async_copy(x_hbm.at[cidx_v], rows_v, sem)
    d2 = pltpu.async_copy(pm_hbm.at[cidx_v], pmc_v, sem2)
    d1.wait()
    d2.wait()
    pltpu.sync_copy(rows_v, xbuf_hbm.at[pl.ds(cbase, CHUNK)])
    pltpu.sync_copy(pmc_v, pms_hbm.at[pl.ds(cbase, CHUNK)])


_NKEEP = KEEP_PAD // CAP


def _ffn_body(xb_ref, pms_ref, xk_ref, pmk_ref, w1_ref, w2_ref, o_ref):
    i = pl.program_id(0)

    @pl.when(i < E)
    def _expert():
        h = jnp.dot(xb_ref[...], w1_ref[0], preferred_element_type=_f32)
        h = jnp.maximum(h, 0.0)
        o = jnp.dot(h, w2_ref[0], preferred_element_type=_f32)
        o_ref[...] = o * pms_ref[...]

    @pl.when(i >= E)
    def _keep():
        o_ref[...] = xk_ref[...] * pmk_ref[...]


_ffn = pl.pallas_call(
    _ffn_body,
    grid=(E + _NKEEP,),
    in_specs=[
        pl.BlockSpec((CAP, D), lambda i: (jnp.minimum(i, E - 1), 0)),
        pl.BlockSpec((CAP, 1), lambda i: (jnp.minimum(i, E - 1), 0)),
        pl.BlockSpec((CAP, D), lambda i: (jnp.maximum(i - E, 0), 0)),
        pl.BlockSpec((CAP, 1), lambda i: (jnp.maximum(i - E, 0), 0)),
        pl.BlockSpec((1, D, DFF), lambda i: (jnp.minimum(i, E - 1), 0, 0)),
        pl.BlockSpec((1, DFF, D), lambda i: (jnp.minimum(i, E - 1), 0, 0)),
    ],
    out_specs=pl.BlockSpec((CAP, D), lambda i: (i, 0)),
    out_shape=jax.ShapeDtypeStruct((SRC_ROWS, D), _f32),
)


@functools.partial(
    pl.kernel,
    out_type=jax.ShapeDtypeStruct((S, D), _f32),
    mesh=_mesh,
    scratch_types=[
        pltpu.VMEM((TOK_W,), _i32),
        pltpu.VMEM((TOK_W, D), _f32),
        pltpu.SemaphoreType.DMA,
    ],
    compiler_params=pltpu.CompilerParams(needs_layout_passes=False),
)
def _combine(src_hbm, big_hbm, out_hbm, idx_v, rows_v, sem):
    wid = lax.axis_index("s") * NC + lax.axis_index("c")
    tbase = wid * TOK_W
    pltpu.sync_copy(src_hbm.at[pl.ds(tbase, TOK_W)], idx_v)
    pltpu.async_copy(big_hbm.at[idx_v], rows_v, sem).wait()
    pltpu.sync_copy(rows_v, out_hbm.at[pl.ds(tbase, TOK_W)])


def kernel(norm_data, Wg, W1, W2):
    x = norm_data.reshape(S, D).astype(_f32)
    wgp = jnp.pad(Wg.astype(_f32), ((0, 0), (0, LANEPAD - E)))
    logits128, pm, idx, onehot = _router(x, wgp)
    src_idx, eidx = _priority(onehot, idx)
    router_logits = logits128[:, :E].reshape(1, S, E)
    src1 = src_idx.reshape(S)
    pm1 = pm.reshape(S)
    xbuf, pm_slot = _dispatch(src1, pm1, x)
    x_pad = jnp.pad(x, ((0, KEEP_PAD - S), (0, 0)))
    pm_pad = jnp.pad(pm, ((0, KEEP_PAD - S), (0, 0)))
    big = _ffn(xbuf, pm_slot.reshape(SLOTS, 1), x_pad, pm_pad, W1, W2)
    out = _combine(src1, big)
    return (out.reshape(1, S, D), router_logits, eidx.reshape(1, S))
